# Initial kernel scaffold; baseline (speedup 1.0000x reference)
import jax
import jax.numpy as jnp
from jax.experimental import pallas as pl


def kernel(u, E_x, E_y, bias, a1, b1, c, d):
    raise NotImplementedError("write your pallas kernel here")


# speedup vs baseline: 1.3656x; 1.3656x over previous
import jax
import jax.numpy as jnp
from jax.experimental import pallas as pl
from jax.experimental.pallas import tpu as pltpu

D_MODEL = 2048
NUM_EXPERTS = 64
TOP_K = 2
SCALE = 2.0
D_HALF = D_MODEL // 2
N_TOKENS = 8192

BLK = 1024
GRID = N_TOKENS // BLK


def _router_body(u_ref, ex_ref, ey_ref, bias_ref, scal_ref,
                 ti_ref, ts_ref, sc_ref, aux_ref, acc_ref):
    i = pl.program_id(0)

    ux = u_ref[:, :D_HALF]
    uy = u_ref[:, D_HALF:]
    x = jnp.tanh(jax.lax.dot(ux, ex_ref[...],
                             preferred_element_type=jnp.float32)) * SCALE
    y = jnp.tanh(jax.lax.dot(uy, ey_ref[...],
                             preferred_element_type=jnp.float32)) * SCALE

    a1 = scal_ref[0, 0]
    b1 = scal_ref[0, 1]
    c = scal_ref[0, 2]
    d = scal_ref[0, 3]
    xa = jnp.abs(x)
    ya = jnp.abs(y)
    s = (xa ** a1 + ya ** b1) * jnp.exp(-(xa ** c + ya ** d)) + bias_ref[...]
    sc_ref[...] = s

    cols = jax.lax.broadcasted_iota(jnp.int32, s.shape, 1)
    m1 = jnp.max(s, axis=1, keepdims=True)
    i1 = jnp.min(jnp.where(s == m1, cols, NUM_EXPERTS), axis=1, keepdims=True)
    masked = jnp.where(cols == i1, -jnp.inf, s)
    m2 = jnp.max(masked, axis=1, keepdims=True)
    i2 = jnp.min(jnp.where(masked == m2, cols, NUM_EXPERTS), axis=1,
                 keepdims=True)
    ts_ref[...] = jnp.concatenate([m1, m2], axis=1)
    ti_ref[...] = jnp.concatenate([i1, i2], axis=1)

    e = jnp.exp(s - m1)
    p = e / jnp.sum(e, axis=1, keepdims=True)
    psum = jnp.sum(p, axis=0, keepdims=True)

    @pl.when(i == 0)
    def _():
        acc_ref[...] = jnp.zeros_like(acc_ref)

    acc_ref[...] += psum

    @pl.when(i == GRID - 1)
    def _():
        mean = acc_ref[...] * (1.0 / N_TOKENS)
        aux_ref[...] = jnp.sum(mean * mean, keepdims=True) * NUM_EXPERTS


def kernel(u, E_x, E_y, bias, a1, b1, c, d):
    bias2 = jnp.reshape(bias, (1, NUM_EXPERTS))
    scal = jnp.stack([jnp.asarray(a1, jnp.float32), jnp.asarray(b1, jnp.float32),
                      jnp.asarray(c, jnp.float32), jnp.asarray(d, jnp.float32)]
                     ).reshape(1, 4)

    topk_i, topk_s, scores, aux = pl.pallas_call(
        _router_body,
        grid=(GRID,),
        in_specs=[
            pl.BlockSpec((BLK, D_MODEL), lambda i: (i, 0)),
            pl.BlockSpec((D_HALF, NUM_EXPERTS), lambda i: (0, 0)),
            pl.BlockSpec((D_HALF, NUM_EXPERTS), lambda i: (0, 0)),
            pl.BlockSpec((1, NUM_EXPERTS), lambda i: (0, 0)),
            pl.BlockSpec(memory_space=pltpu.SMEM),
        ],
        out_specs=[
            pl.BlockSpec((BLK, TOP_K), lambda i: (i, 0)),
            pl.BlockSpec((BLK, TOP_K), lambda i: (i, 0)),
            pl.BlockSpec((BLK, NUM_EXPERTS), lambda i: (i, 0)),
            pl.BlockSpec((1, 1), lambda i: (0, 0)),
        ],
        out_shape=[
            jax.ShapeDtypeStruct((N_TOKENS, TOP_K), jnp.int32),
            jax.ShapeDtypeStruct((N_TOKENS, TOP_K), jnp.float32),
            jax.ShapeDtypeStruct((N_TOKENS, NUM_EXPERTS), jnp.float32),
            jax.ShapeDtypeStruct((1, 1), jnp.float32),
        ],
        scratch_shapes=[pltpu.VMEM((1, NUM_EXPERTS), jnp.float32)],
    )(u, E_x, E_y, bias2, scal)

    return (topk_i, topk_s, scores, aux[0, 0])
